# baseline (speedup 1.0000x reference)
import functools

import jax
import jax.numpy as jnp
from jax import lax
from jax.experimental import pallas as pl
from jax.experimental.pallas import tpu as pltpu
from jax.experimental.pallas import tpu_sc as plsc

_N = 10000
_E = 320000
_D = 128

_NC = 2
_NS = 16
_HD = _D // _NC
_EPT = _E // _NS
_C = 80
_NK = _EPT // _C
_RPT = _N // _NS
_ZR = 25
_LANES = 16
_NBUF = 5
_L = _NBUF - 2


def _scale_chunk(src, dst, vals, k):
    def body(g, carry):
        vgrp = vals[k, pl.ds(g * _LANES, _LANES)]
        for u in range(_LANES):
            e = g * _LANES + u
            vv = jnp.full((_LANES,), vgrp[u], jnp.float32)
            for j in range(_HD // _LANES):
                sl = pl.ds(j * _LANES, _LANES)
                dst[e, sl] = src[e, sl].astype(jnp.float32) * vv
        return carry

    lax.fori_loop(0, _C // _LANES, body, 0)


_sc_mesh = plsc.VectorSubcoreMesh(
    core_axis_name="c", subcore_axis_name="s", num_cores=_NC, num_subcores=_NS
)


@functools.partial(
    pl.kernel,
    out_type=jax.ShapeDtypeStruct((_NC, _NS, _RPT, _HD), jnp.float32),
    mesh=_sc_mesh,
    compiler_params=pltpu.CompilerParams(use_tc_tiling_on_sc=False),
    scratch_types=[
        pltpu.VMEM((2, _NBUF, _C), jnp.int32),
        pltpu.VMEM((2, _NBUF, _C), jnp.int32),
        pltpu.VMEM((_NK, _C), jnp.float32),
        [pltpu.VMEM((_C, _HD), jnp.bfloat16) for _ in range(_NBUF)],
        [pltpu.VMEM((_C, _HD), jnp.float32) for _ in range(_NBUF)],
        pltpu.VMEM((_ZR, _HD), jnp.float32),
        pltpu.VMEM_SHARED((_N, _HD), jnp.float32),
        [pltpu.SemaphoreType.DMA for _ in range(_NBUF)],
        [pltpu.SemaphoreType.DMA for _ in range(_NBUF)],
        pltpu.SemaphoreType.DMA,
        pltpu.SemaphoreType.DMA,
        pltpu.SemaphoreType.DMA,
    ],
)
def _spmm(rows_hbm, cols_hbm, vals_hbm, h_hbm, out_hbm,
          ridx, cidx, vals, gbufs, sbufs, zbuf, acc, gsems, ssems, fsem,
          zsem, vsem):
    cid = lax.axis_index("c")
    sid = lax.axis_index("s")

    table = h_hbm.at[cid]
    _NG = _NK // _NBUF

    def fill(g, slot):
        sl = pl.ds(g * _NBUF, _NBUF)
        pltpu.async_copy(cols_hbm.at[sid, sl], cidx.at[slot], fsem)
        pltpu.async_copy(rows_hbm.at[sid, sl], ridx.at[slot], fsem)

    def fwait():
        sl = pl.ds(0, _NBUF)
        pltpu.make_async_copy(cols_hbm.at[sid, sl], cidx.at[0], fsem).wait()
        pltpu.make_async_copy(rows_hbm.at[sid, sl], ridx.at[0], fsem).wait()

    pltpu.async_copy(vals_hbm.at[sid], vals, vsem)

    def zbody(r, carry):
        zrow = jnp.zeros((_LANES,), jnp.float32)
        for j in range(_HD // _LANES):
            zbuf[r, pl.ds(j * _LANES, _LANES)] = zrow
        return carry

    lax.fori_loop(0, _ZR, zbody, 0)

    def zissue(i, carry):
        pltpu.async_copy(zbuf, acc.at[pl.ds(sid * _RPT + i * _ZR, _ZR)], zsem)
        return carry

    def zdrain(i, carry):
        pltpu.make_async_copy(zbuf, acc.at[pl.ds(sid * _RPT, _ZR)], zsem).wait()
        return carry

    lax.fori_loop(0, _RPT // _ZR, zissue, 0)
    fill(0, 0)
    fill(1, 1)
    lax.fori_loop(0, _RPT // _ZR, zdrain, 0)
    plsc.subcore_barrier()

    def gather(slot, row, b):
        pltpu.async_copy(table.at[cidx.at[slot, row]], gbufs[b], gsems[b])

    def gwait(b):
        pltpu.make_async_copy(table.at[cidx.at[0, 0]], gbufs[b], gsems[b]).wait()

    def work(k, slot, row, b):
        _scale_chunk(gbufs[b], sbufs[b], vals, k)
        pltpu.async_copy(sbufs[b], acc.at[ridx.at[slot, row]], ssems[b], add=True)

    def swait(b):
        pltpu.make_async_copy(sbufs[b], acc.at[ridx.at[0, 0]], ssems[b]).wait()

    fwait()
    fwait()
    for i in range(_L):
        gather(0, i, i)
    pltpu.make_async_copy(vals_hbm.at[sid], vals, vsem).wait()
    for b in range(_NBUF):
        gwait(b)
        work(b, 0, b, b)
        bn = (b + _L) % _NBUF
        if b >= 2:
            swait(bn)
        g2, r2 = divmod(b + _L, _NBUF)
        gather(g2, r2, bn)

    def lbody(m, carry):
        slot = lax.rem(m, 2)
        nslot = lax.rem(m + 1, 2)
        for b in range(_NBUF):
            k = m * _NBUF + b
            gwait(b)
            work(k, slot, b, b)
            bn = (b + _L) % _NBUF
            swait(bn)
            if b == 1:

                @pl.when(m + 1 < _NG)
                def _():
                    fill(m + 1, nslot)

            if b < 2:
                gather(slot, b + _L, bn)
            else:

                @pl.when(k + _L < _NK)
                def _():
                    if b == 2:
                        fwait()
                    gather(nslot, b - 2, bn)

        return carry

    lax.fori_loop(1, _NG, lbody, 0)
    swait((_NK - 2) % _NBUF)
    swait((_NK - 1) % _NBUF)

    plsc.subcore_barrier()
    pltpu.sync_copy(acc.at[pl.ds(sid * _RPT, _RPT)], out_hbm.at[cid, sid])


_BLK = 2000


def _dense1_body(x_ref, w1_ref, b1_ref, wg_ref, o_ref):
    h = jnp.dot(x_ref[...], w1_ref[...], preferred_element_type=jnp.float32)
    h = h + b1_ref[...]
    h = 1.0 / (1.0 + jnp.exp(-h))
    h = jnp.dot(h, wg_ref[...], preferred_element_type=jnp.float32)
    h = h.astype(jnp.bfloat16)
    o_ref[0] = h[:, :_HD]
    o_ref[1] = h[:, _HD:]


def _dense1(x, w1, b1, wg):
    return pl.pallas_call(
        _dense1_body,
        grid=(_N // _BLK,),
        in_specs=[
            pl.BlockSpec((_BLK, _D), lambda i: (i, 0)),
            pl.BlockSpec((_D, _D), lambda i: (0, 0)),
            pl.BlockSpec((1, _D), lambda i: (0, 0)),
            pl.BlockSpec((_D, _D), lambda i: (0, 0)),
        ],
        out_specs=pl.BlockSpec((_NC, _BLK, _HD), lambda i: (0, i, 0)),
        out_shape=jax.ShapeDtypeStruct((_NC, _N, _HD), jnp.bfloat16),
    )(x, w1, b1, wg)


def _dense_mid_body(p_ref, b_ref, w_ref, o_ref):
    s = jnp.concatenate([p_ref[0], p_ref[1]], axis=1) + b_ref[...]
    s = jnp.maximum(s, 0.0)
    h = jnp.dot(s, w_ref[...], preferred_element_type=jnp.float32)
    h = h.astype(jnp.bfloat16)
    o_ref[0] = h[:, :_HD]
    o_ref[1] = h[:, _HD:]


def _dense_mid(p, b, w):
    return pl.pallas_call(
        _dense_mid_body,
        grid=(_N // _BLK,),
        in_specs=[
            pl.BlockSpec((_NC, _BLK, _HD), lambda i: (0, i, 0)),
            pl.BlockSpec((1, _D), lambda i: (0, 0)),
            pl.BlockSpec((_D, _D), lambda i: (0, 0)),
        ],
        out_specs=pl.BlockSpec((_NC, _BLK, _HD), lambda i: (0, i, 0)),
        out_shape=jax.ShapeDtypeStruct((_NC, _N, _HD), jnp.bfloat16),
    )(p, b, w)


def _dense_out_body(p_ref, b_ref, w_ref, bo_ref, o_ref):
    s = jnp.concatenate([p_ref[0], p_ref[1]], axis=1) + b_ref[...]
    s = jnp.maximum(s, 0.0)
    o_ref[...] = (
        jnp.dot(s, w_ref[...], preferred_element_type=jnp.float32) + bo_ref[...]
    )


def _dense_out(p, b, w, bo):
    return pl.pallas_call(
        _dense_out_body,
        grid=(_N // _BLK,),
        in_specs=[
            pl.BlockSpec((_NC, _BLK, _HD), lambda i: (0, i, 0)),
            pl.BlockSpec((1, _D), lambda i: (0, 0)),
            pl.BlockSpec((_D, _D), lambda i: (0, 0)),
            pl.BlockSpec((1, _D), lambda i: (0, 0)),
        ],
        out_specs=pl.BlockSpec((_BLK, _D), lambda i: (i, 0)),
        out_shape=jax.ShapeDtypeStruct((_N, _D), jnp.float32),
    )(p, b, w, bo)


def kernel(features, edge_index, adj_values, W1, b1, Wg1, bg1, Wg2, bg2, W2, b2):
    ei = edge_index.astype(jnp.int32)
    rows3 = ei[0].reshape(_NS, _NK, _C)
    cols3 = ei[1].reshape(_NS, _NK, _C)
    vals3 = adj_values.reshape(_NS, _NK, _C)

    h1 = _dense1(features, W1, b1.reshape(1, _D), Wg1)
    p1 = _spmm(rows3, cols3, vals3, h1).reshape(_NC, _N, _HD)
    h2 = _dense_mid(p1, bg1.reshape(1, _D), Wg2)
    p2 = _spmm(rows3, cols3, vals3, h2).reshape(_NC, _N, _HD)
    x = _dense_out(p2, bg2.reshape(1, _D), W2, b2.reshape(1, _D))
    return x


# speedup vs baseline: 3.3309x; 3.3309x over previous
import functools

import jax
import jax.numpy as jnp
from jax import lax
from jax.experimental import pallas as pl
from jax.experimental.pallas import tpu as pltpu
from jax.experimental.pallas import tpu_sc as plsc

_N = 10000
_E = 320000
_D = 128

_NC = 2
_NS = 16
_HD = _D // _NC
_EPT = _E // _NS
_C = 80
_NK = _EPT // _C
_RPT = _N // _NS
_ZR = 25
_LANES = 16
_NBUF = 5
_L = _NBUF - 2


def _scale_chunk(src, dst, vals, k):
    def body(g, carry):
        vgrp = vals[k, pl.ds(g * _LANES, _LANES)]
        for u in range(_LANES):
            e = g * _LANES + u
            vv = jnp.full((_LANES,), vgrp[u], jnp.float32)
            for j in range(_HD // _LANES):
                sl = pl.ds(j * _LANES, _LANES)
                dst[e, sl] = src[e, sl] * vv
        return carry

    lax.fori_loop(0, _C // _LANES, body, 0)


_sc_mesh = plsc.VectorSubcoreMesh(
    core_axis_name="c", subcore_axis_name="s", num_cores=_NC, num_subcores=_NS
)


@functools.partial(
    pl.kernel,
    out_type=jax.ShapeDtypeStruct((_NC, _NS, _RPT, _HD), jnp.float32),
    mesh=_sc_mesh,
    compiler_params=pltpu.CompilerParams(use_tc_tiling_on_sc=False),
    scratch_types=[
        pltpu.VMEM((2, _NBUF, _C), jnp.int32),
        pltpu.VMEM((2, _NBUF, _C), jnp.int32),
        pltpu.VMEM((_NK, _C), jnp.float32),
        [pltpu.VMEM((_C, _HD), jnp.float32) for _ in range(_NBUF)],
        [pltpu.VMEM((_C, _HD), jnp.float32) for _ in range(_NBUF)],
        pltpu.VMEM((_ZR, _HD), jnp.float32),
        pltpu.VMEM_SHARED((_N, _HD), jnp.float32),
        [pltpu.SemaphoreType.DMA for _ in range(_NBUF)],
        [pltpu.SemaphoreType.DMA for _ in range(_NBUF)],
        pltpu.SemaphoreType.DMA,
        pltpu.SemaphoreType.DMA,
        pltpu.SemaphoreType.DMA,
    ],
)
def _spmm(rows_hbm, cols_hbm, vals_hbm, h_hbm, out_hbm,
          ridx, cidx, vals, gbufs, sbufs, zbuf, acc, gsems, ssems, fsem,
          zsem, vsem):
    cid = lax.axis_index("c")
    sid = lax.axis_index("s")

    table = h_hbm.at[cid]
    _NG = _NK // _NBUF

    def fill(g, slot):
        sl = pl.ds(g * _NBUF, _NBUF)
        pltpu.async_copy(cols_hbm.at[sid, sl], cidx.at[slot], fsem)
        pltpu.async_copy(rows_hbm.at[sid, sl], ridx.at[slot], fsem)

    def fwait():
        sl = pl.ds(0, _NBUF)
        pltpu.make_async_copy(cols_hbm.at[sid, sl], cidx.at[0], fsem).wait()
        pltpu.make_async_copy(rows_hbm.at[sid, sl], ridx.at[0], fsem).wait()

    pltpu.async_copy(vals_hbm.at[sid], vals, vsem)

    def zbody(r, carry):
        zrow = jnp.zeros((_LANES,), jnp.float32)
        for j in range(_HD // _LANES):
            zbuf[r, pl.ds(j * _LANES, _LANES)] = zrow
        return carry

    lax.fori_loop(0, _ZR, zbody, 0)

    def zissue(i, carry):
        pltpu.async_copy(zbuf, acc.at[pl.ds(sid * _RPT + i * _ZR, _ZR)], zsem)
        return carry

    def zdrain(i, carry):
        pltpu.make_async_copy(zbuf, acc.at[pl.ds(sid * _RPT, _ZR)], zsem).wait()
        return carry

    lax.fori_loop(0, _RPT // _ZR, zissue, 0)
    fill(0, 0)
    fill(1, 1)
    lax.fori_loop(0, _RPT // _ZR, zdrain, 0)
    plsc.subcore_barrier()

    def gather(slot, row, b):
        pltpu.async_copy(table.at[cidx.at[slot, row]], gbufs[b], gsems[b])

    def gwait(b):
        pltpu.make_async_copy(table.at[cidx.at[0, 0]], gbufs[b], gsems[b]).wait()

    def work(k, slot, row, b):
        _scale_chunk(gbufs[b], sbufs[b], vals, k)
        pltpu.async_copy(sbufs[b], acc.at[ridx.at[slot, row]], ssems[b], add=True)

    def swait(b):
        pltpu.make_async_copy(sbufs[b], acc.at[ridx.at[0, 0]], ssems[b]).wait()

    fwait()
    fwait()
    for i in range(_L):
        gather(0, i, i)
    pltpu.make_async_copy(vals_hbm.at[sid], vals, vsem).wait()
    for b in range(_NBUF):
        gwait(b)
        bn = (b + _L) % _NBUF
        g2, r2 = divmod(b + _L, _NBUF)
        gather(g2, r2, bn)
        if b >= 2:
            swait(bn)
        work(b, 0, b, b)

    def lbody(m, carry):
        slot = lax.rem(m, 2)
        nslot = lax.rem(m + 1, 2)
        for b in range(_NBUF):
            k = m * _NBUF + b
            gwait(b)
            bn = (b + _L) % _NBUF
            if b == 0:

                @pl.when(m + 1 < _NG)
                def _():
                    fill(m + 1, nslot)

            if b < 2:
                gather(slot, b + _L, bn)
            else:

                @pl.when(k + _L < _NK)
                def _():
                    if b == 2:
                        fwait()
                    gather(nslot, b - 2, bn)

            swait(bn)
            work(k, slot, b, b)

        return carry

    lax.fori_loop(1, _NG, lbody, 0)
    swait((_NK - 2) % _NBUF)
    swait((_NK - 1) % _NBUF)

    plsc.subcore_barrier()
    pltpu.sync_copy(acc.at[pl.ds(sid * _RPT, _RPT)], out_hbm.at[cid, sid])


_BLK = 2000


def _dense1_body(x_ref, w1_ref, b1_ref, wg_ref, o_ref):
    h = jnp.dot(x_ref[...], w1_ref[...], preferred_element_type=jnp.float32)
    h = h + b1_ref[...]
    h = 1.0 / (1.0 + jnp.exp(-h))
    h = jnp.dot(h, wg_ref[...], preferred_element_type=jnp.float32)
    o_ref[0] = h[:, :_HD]
    o_ref[1] = h[:, _HD:]


def _dense1(x, w1, b1, wg):
    return pl.pallas_call(
        _dense1_body,
        grid=(_N // _BLK,),
        in_specs=[
            pl.BlockSpec((_BLK, _D), lambda i: (i, 0)),
            pl.BlockSpec((_D, _D), lambda i: (0, 0)),
            pl.BlockSpec((1, _D), lambda i: (0, 0)),
            pl.BlockSpec((_D, _D), lambda i: (0, 0)),
        ],
        out_specs=pl.BlockSpec((_NC, _BLK, _HD), lambda i: (0, i, 0)),
        out_shape=jax.ShapeDtypeStruct((_NC, _N, _HD), jnp.float32),
    )(x, w1, b1, wg)


def _dense_mid_body(p_ref, b_ref, w_ref, o_ref):
    s = jnp.concatenate([p_ref[0], p_ref[1]], axis=1) + b_ref[...]
    s = jnp.maximum(s, 0.0)
    h = jnp.dot(s, w_ref[...], preferred_element_type=jnp.float32)
    o_ref[0] = h[:, :_HD]
    o_ref[1] = h[:, _HD:]


def _dense_mid(p, b, w):
    return pl.pallas_call(
        _dense_mid_body,
        grid=(_N // _BLK,),
        in_specs=[
            pl.BlockSpec((_NC, _BLK, _HD), lambda i: (0, i, 0)),
            pl.BlockSpec((1, _D), lambda i: (0, 0)),
            pl.BlockSpec((_D, _D), lambda i: (0, 0)),
        ],
        out_specs=pl.BlockSpec((_NC, _BLK, _HD), lambda i: (0, i, 0)),
        out_shape=jax.ShapeDtypeStruct((_NC, _N, _HD), jnp.float32),
    )(p, b, w)


def _dense_out_body(p_ref, b_ref, w_ref, bo_ref, o_ref):
    s = jnp.concatenate([p_ref[0], p_ref[1]], axis=1) + b_ref[...]
    s = jnp.maximum(s, 0.0)
    o_ref[...] = (
        jnp.dot(s, w_ref[...], preferred_element_type=jnp.float32) + bo_ref[...]
    )


def _dense_out(p, b, w, bo):
    return pl.pallas_call(
        _dense_out_body,
        grid=(_N // _BLK,),
        in_specs=[
            pl.BlockSpec((_NC, _BLK, _HD), lambda i: (0, i, 0)),
            pl.BlockSpec((1, _D), lambda i: (0, 0)),
            pl.BlockSpec((_D, _D), lambda i: (0, 0)),
            pl.BlockSpec((1, _D), lambda i: (0, 0)),
        ],
        out_specs=pl.BlockSpec((_BLK, _D), lambda i: (i, 0)),
        out_shape=jax.ShapeDtypeStruct((_N, _D), jnp.float32),
    )(p, b, w, bo)


def kernel(features, edge_index, adj_values, W1, b1, Wg1, bg1, Wg2, bg2, W2, b2):
    ei = edge_index.astype(jnp.int32)
    rows3 = ei[0].reshape(_NS, _NK, _C)
    cols3 = ei[1].reshape(_NS, _NK, _C)
    vals3 = adj_values.reshape(_NS, _NK, _C)

    h1 = _dense1(features, W1, b1.reshape(1, _D), Wg1)
    p1 = _spmm(rows3, cols3, vals3, h1).reshape(_NC, _N, _HD)
    h2 = _dense_mid(p1, bg1.reshape(1, _D), Wg2)
    p2 = _spmm(rows3, cols3, vals3, h2).reshape(_NC, _N, _HD)
    x = _dense_out(p2, bg2.reshape(1, _D), W2, b2.reshape(1, _D))
    return x
